# baseline (speedup 1.0000x reference)
import functools
import jax
import jax.numpy as jnp
from jax import lax
from jax.experimental import pallas as pl
from jax.experimental.pallas import tpu as pltpu
from jax.experimental.pallas import tpu_sc as plsc

_NW = 32
_R = 64
_GB = 79
_G = 16 * _GB
_T = (_G + _NW - 1) // _NW


def _sc_post(lg_hbm, bx_hbm, cls_hbm, ts_hbm,
             sc_out, lb_out, bx_out, cl_out,
             lgA, lgB, bxA, bxB,
             osA, osB, olA, olB, obA, obB,
             tsv, clsv, cll,
             semA, semB, osemA, osemB):
    wid = lax.axis_index("s") * 2 + lax.axis_index("c")
    iota = lax.iota(jnp.int32, 16)
    base4 = iota * 4

    @pl.when(wid == 0)
    def _():
        pltpu.sync_copy(cls_hbm, clsv)
        m0 = plsc.load_gather(clsv, [iota * 10])
        lab0 = jnp.zeros((16,), jnp.int32)

        def cbody(c, carry):
            m, lab = carry
            v = plsc.load_gather(clsv, [iota * 10 + c])
            upd = v > m
            return jnp.where(upd, v, m), jnp.where(upd, c, lab)

        _, lab = lax.fori_loop(1, 10, cbody, (m0, lab0))
        cll[...] = lab
        pltpu.sync_copy(cll, cl_out)

    pltpu.sync_copy(ts_hbm, tsv)

    def g_to_br(g):
        b = g // _GB
        t = g - b * _GB
        r = jnp.where(t == _GB - 1, 5000 - _R, t * _R)
        return b, pl.multiple_of(r, 8)

    def start_in(g, lg_buf, bx_buf, sem):
        b, r = g_to_br(g)
        pltpu.async_copy(lg_hbm.at[b, pl.ds(r, _R), :], lg_buf, sem)
        pltpu.async_copy(bx_hbm.at[pl.ds((b * 5000 + r) * 4, _R * 4)],
                         bx_buf, sem)

    def wait_in(lg_buf, bx_buf, sem):
        pltpu.make_async_copy(lg_hbm.at[0, pl.ds(0, _R), :], lg_buf,
                              sem).wait()
        pltpu.make_async_copy(bx_hbm.at[pl.ds(0, _R * 4)], bx_buf,
                              sem).wait()

    def wait_out(os_, ol_, ob_, osem):
        pltpu.make_async_copy(sc_out.at[pl.ds(0, _R)], os_, osem).wait()
        pltpu.make_async_copy(lb_out.at[pl.ds(0, _R)], ol_, osem).wait()
        pltpu.make_async_copy(bx_out.at[pl.ds(0, _R * 4)], ob_,
                              osem).wait()

    start_in(wid, lgA, bxA, semA)
    start_in(wid + _NW, lgB, bxB, semB)

    def slot(i, sl, lg_buf, bx_buf, os_, ol_, ob_, sem, osem):
        tt = 2 * i + sl
        g = wid + _NW * tt

        @pl.when(g < _G)
        def _():
            b, r = g_to_br(g)
            row0 = b * 5000 + r
            wait_in(lg_buf, bx_buf, sem)

            @pl.when(tt >= 2)
            def _():
                wait_out(os_, ol_, ob_, osem)

            zero = jnp.zeros((16,), jnp.int32)
            rows = tuple(iota + 16 * k for k in range(4))
            m0 = [plsc.load_gather(lg_buf, [rw, zero]) for rw in rows]
            s0 = [jnp.exp(m) for m in m0]
            lab0 = [zero] * 4

            def body(c, carry):
                ms, labs, ss = carry
                col = zero + c
                nm, nl, ns = [], [], []
                for k in range(4):
                    v = plsc.load_gather(lg_buf, [rows[k], col])
                    upd = v > ms[k]
                    nm.append(jnp.where(upd, v, ms[k]))
                    nl.append(jnp.where(upd, c, labs[k]))
                    ns.append(ss[k] + jnp.exp(v))
                return tuple(nm), tuple(nl), tuple(ns)

            ms, labs, ss = lax.fori_loop(
                1, 91, body, (tuple(m0), tuple(lab0), tuple(s0)), unroll=7)
            for k in range(4):
                os_[pl.ds(16 * k, 16)] = jnp.exp(ms[k]) / ss[k]
                ol_[pl.ds(16 * k, 16)] = labs[k]

            shv = plsc.load_gather(tsv, [zero + 2 * b]).astype(jnp.float32)
            swv = plsc.load_gather(tsv, [zero + 2 * b + 1]).astype(
                jnp.float32)
            one = jnp.float32(1.0)
            zf = jnp.float32(0.0)
            for k in range(4):
                b4 = base4 + 64 * k
                cx = plsc.load_gather(bx_buf, [b4])
                cy = plsc.load_gather(bx_buf, [b4 + 1])
                w = plsc.load_gather(bx_buf, [b4 + 2])
                h = plsc.load_gather(bx_buf, [b4 + 3])
                x0 = jnp.clip(cx - 0.5 * w, zf, one) * swv
                y0 = jnp.clip(cy - 0.5 * h, zf, one) * shv
                x1 = jnp.clip(cx + 0.5 * w, zf, one) * swv
                y1 = jnp.clip(cy + 0.5 * h, zf, one) * shv
                plsc.store_scatter(ob_, [b4], x0)
                plsc.store_scatter(ob_, [b4 + 1], y0)
                plsc.store_scatter(ob_, [b4 + 2], x1)
                plsc.store_scatter(ob_, [b4 + 3], y1)

            pltpu.async_copy(os_, sc_out.at[pl.ds(row0, _R)], osem)
            pltpu.async_copy(ol_, lb_out.at[pl.ds(row0, _R)], osem)
            pltpu.async_copy(ob_, bx_out.at[pl.ds(row0 * 4, _R * 4)], osem)

            g2 = g + 2 * _NW

            @pl.when(g2 < _G)
            def _():
                start_in(g2, lg_buf, bx_buf, sem)

    def lbody(i, _):
        slot(i, 0, lgA, bxA, osA, olA, obA, semA, osemA)
        slot(i, 1, lgB, bxB, osB, olB, obB, semB, osemB)
        return 0

    lax.fori_loop(0, _T // 2, lbody, 0)

    wait_out(osA, olA, obA, osemA)
    wait_out(osB, olB, obB, osemB)


def kernel(pred_logits, pred_boxes, cls_logits, target_sizes):
    nb, nq, nc = pred_logits.shape
    mesh = plsc.VectorSubcoreMesh(core_axis_name="c", subcore_axis_name="s")
    fn = functools.partial(
        pl.kernel,
        mesh=mesh,
        compiler_params=pltpu.CompilerParams(needs_layout_passes=False),
        out_type=[
            jax.ShapeDtypeStruct((nb * nq,), jnp.float32),
            jax.ShapeDtypeStruct((nb * nq,), jnp.int32),
            jax.ShapeDtypeStruct((nb * nq * 4,), jnp.float32),
            jax.ShapeDtypeStruct((nb,), jnp.int32),
        ],
        scratch_types=[
            pltpu.VMEM((_R, 91), jnp.float32),
            pltpu.VMEM((_R, 91), jnp.float32),
            pltpu.VMEM((_R * 4,), jnp.float32),
            pltpu.VMEM((_R * 4,), jnp.float32),
            pltpu.VMEM((_R,), jnp.float32),
            pltpu.VMEM((_R,), jnp.float32),
            pltpu.VMEM((_R,), jnp.int32),
            pltpu.VMEM((_R,), jnp.int32),
            pltpu.VMEM((_R * 4,), jnp.float32),
            pltpu.VMEM((_R * 4,), jnp.float32),
            pltpu.VMEM((32,), jnp.int32),
            pltpu.VMEM((160,), jnp.float32),
            pltpu.VMEM((16,), jnp.int32),
            pltpu.SemaphoreType.DMA,
            pltpu.SemaphoreType.DMA,
            pltpu.SemaphoreType.DMA,
            pltpu.SemaphoreType.DMA,
        ],
    )(_sc_post)
    scores, labels, boxes, cls2 = fn(
        pred_logits, pred_boxes.reshape(-1), cls_logits.reshape(-1),
        target_sizes.reshape(-1))
    return (scores.reshape(nb, nq), labels.reshape(nb, nq),
            boxes.reshape(nb, nq, 4), cls2)


# speedup vs baseline: 1.4915x; 1.4915x over previous
import functools
import jax
import jax.numpy as jnp
from jax import lax
from jax.experimental import pallas as pl
from jax.experimental.pallas import tpu as pltpu
from jax.experimental.pallas import tpu_sc as plsc

_QPAD = 5120

_NW = 32
_R = 64
_GB = 79
_G = 16 * _GB
_T = (_G + _NW - 1) // _NW


def _sc_boxes(bx_hbm, cls_hbm, ts_hbm, bx_out, cl_out,
              bxA, bxB, obA, obB, tsv, clsv, cll,
              semA, semB, osemA, osemB):
    wid = lax.axis_index("s") * 2 + lax.axis_index("c")
    iota = lax.iota(jnp.int32, 16)
    base4 = iota * 4

    @pl.when(wid == 0)
    def _():
        pltpu.sync_copy(cls_hbm, clsv)
        m0 = plsc.load_gather(clsv, [iota * 10])
        lab0 = jnp.zeros((16,), jnp.int32)

        def cbody(c, carry):
            m, lab = carry
            v = plsc.load_gather(clsv, [iota * 10 + c])
            upd = v > m
            return jnp.where(upd, v, m), jnp.where(upd, c, lab)

        _, lab = lax.fori_loop(1, 10, cbody, (m0, lab0))
        cll[...] = lab
        pltpu.sync_copy(cll, cl_out)

    pltpu.sync_copy(ts_hbm, tsv)

    def g_to_br(g):
        b = g // _GB
        t = g - b * _GB
        r = jnp.where(t == _GB - 1, 5000 - _R, t * _R)
        return b, pl.multiple_of(r, 8)

    def start_in(g, bx_buf, sem):
        b, r = g_to_br(g)
        pltpu.async_copy(bx_hbm.at[pl.ds((b * 5000 + r) * 4, _R * 4)],
                         bx_buf, sem)

    def wait_in(bx_buf, sem):
        pltpu.make_async_copy(bx_hbm.at[pl.ds(0, _R * 4)], bx_buf,
                              sem).wait()

    def wait_out(ob_, osem):
        pltpu.make_async_copy(bx_out.at[pl.ds(0, _R * 4)], ob_,
                              osem).wait()

    start_in(wid, bxA, semA)
    start_in(wid + _NW, bxB, semB)

    def slot(i, sl, bx_buf, ob_, sem, osem):
        tt = 2 * i + sl
        g = wid + _NW * tt

        @pl.when(g < _G)
        def _():
            b, r = g_to_br(g)
            row0 = b * 5000 + r
            wait_in(bx_buf, sem)

            @pl.when(tt >= 2)
            def _():
                wait_out(ob_, osem)

            zero = jnp.zeros((16,), jnp.int32)
            shv = plsc.load_gather(tsv, [zero + 2 * b]).astype(jnp.float32)
            swv = plsc.load_gather(tsv, [zero + 2 * b + 1]).astype(
                jnp.float32)
            one = jnp.float32(1.0)
            zf = jnp.float32(0.0)
            for k in range(4):
                b4 = base4 + 64 * k
                cx = plsc.load_gather(bx_buf, [b4])
                cy = plsc.load_gather(bx_buf, [b4 + 1])
                w = plsc.load_gather(bx_buf, [b4 + 2])
                h = plsc.load_gather(bx_buf, [b4 + 3])
                x0 = jnp.clip(cx - 0.5 * w, zf, one) * swv
                y0 = jnp.clip(cy - 0.5 * h, zf, one) * shv
                x1 = jnp.clip(cx + 0.5 * w, zf, one) * swv
                y1 = jnp.clip(cy + 0.5 * h, zf, one) * shv
                plsc.store_scatter(ob_, [b4], x0)
                plsc.store_scatter(ob_, [b4 + 1], y0)
                plsc.store_scatter(ob_, [b4 + 2], x1)
                plsc.store_scatter(ob_, [b4 + 3], y1)

            pltpu.async_copy(ob_, bx_out.at[pl.ds(row0 * 4, _R * 4)], osem)

            g2 = g + 2 * _NW

            @pl.when(g2 < _G)
            def _():
                start_in(g2, bx_buf, sem)

    def lbody(i, _):
        slot(i, 0, bxA, obA, semA, osemA)
        slot(i, 1, bxB, obB, semB, osemB)
        return 0

    lax.fori_loop(0, _T // 2, lbody, 0)
    wait_out(obA, osemA)
    wait_out(obB, osemB)


def _tc_body(logits_ref, scores_ref, labels_ref):
    nq = logits_ref.shape[1]
    pad = _QPAD - nq
    xt = logits_ref[0].T
    c_iota = jax.lax.broadcasted_iota(jnp.int32, xt.shape, 0)
    m = jnp.max(xt, axis=0)
    labels = jnp.min(jnp.where(xt == m[None, :], c_iota, 91), axis=0)
    s = jnp.sum(jnp.exp(xt), axis=0)
    scores = jnp.exp(m) / s
    scores_ref[0] = jnp.concatenate(
        [scores, jnp.zeros((pad,), jnp.float32)]).reshape(1, _QPAD)
    labels_ref[0] = jnp.concatenate(
        [labels, jnp.zeros((pad,), jnp.int32)]).reshape(1, _QPAD)


def kernel(pred_logits, pred_boxes, cls_logits, target_sizes):
    nb, nq, nc = pred_logits.shape

    mesh = plsc.VectorSubcoreMesh(core_axis_name="c", subcore_axis_name="s")
    sc_fn = functools.partial(
        pl.kernel,
        mesh=mesh,
        compiler_params=pltpu.CompilerParams(needs_layout_passes=False),
        out_type=[
            jax.ShapeDtypeStruct((nb * nq * 4,), jnp.float32),
            jax.ShapeDtypeStruct((nb,), jnp.int32),
        ],
        scratch_types=[
            pltpu.VMEM((_R * 4,), jnp.float32),
            pltpu.VMEM((_R * 4,), jnp.float32),
            pltpu.VMEM((_R * 4,), jnp.float32),
            pltpu.VMEM((_R * 4,), jnp.float32),
            pltpu.VMEM((32,), jnp.int32),
            pltpu.VMEM((160,), jnp.float32),
            pltpu.VMEM((16,), jnp.int32),
            pltpu.SemaphoreType.DMA,
            pltpu.SemaphoreType.DMA,
            pltpu.SemaphoreType.DMA,
            pltpu.SemaphoreType.DMA,
        ],
    )(_sc_boxes)
    boxes, cls2 = sc_fn(pred_boxes.reshape(-1), cls_logits.reshape(-1),
                        target_sizes.reshape(-1))

    scores, labels = pl.pallas_call(
        _tc_body,
        grid=(nb,),
        in_specs=[pl.BlockSpec((1, nq, nc), lambda i: (i, 0, 0))],
        out_specs=[
            pl.BlockSpec((1, 1, _QPAD), lambda i: (i, 0, 0)),
            pl.BlockSpec((1, 1, _QPAD), lambda i: (i, 0, 0)),
        ],
        out_shape=[
            jax.ShapeDtypeStruct((nb, 1, _QPAD), jnp.float32),
            jax.ShapeDtypeStruct((nb, 1, _QPAD), jnp.int32),
        ],
    )(pred_logits)

    return (scores[:, 0, :nq], labels[:, 0, :nq],
            boxes.reshape(nb, nq, 4), cls2)


# speedup vs baseline: 1.4918x; 1.0002x over previous
import functools
import jax
import jax.numpy as jnp
from jax import lax
from jax.experimental import pallas as pl
from jax.experimental.pallas import tpu as pltpu
from jax.experimental.pallas import tpu_sc as plsc

_QPAD = 5120

_NW = 32
_R = 64
_GB = 79
_G = 16 * _GB
_T = (_G + _NW - 1) // _NW


def _sc_boxes(bx_hbm, cls_hbm, ts_hbm, bx_out, cl_out,
              bxA, bxB, obA, obB, tsv, clsv, cll,
              semA, semB, osemA, osemB):
    wid = lax.axis_index("s") * 2 + lax.axis_index("c")
    iota = lax.iota(jnp.int32, 16)
    base4 = iota * 4

    @pl.when(wid == 0)
    def _():
        pltpu.sync_copy(cls_hbm, clsv)
        m0 = plsc.load_gather(clsv, [iota * 10])
        lab0 = jnp.zeros((16,), jnp.int32)

        def cbody(c, carry):
            m, lab = carry
            v = plsc.load_gather(clsv, [iota * 10 + c])
            upd = v > m
            return jnp.where(upd, v, m), jnp.where(upd, c, lab)

        _, lab = lax.fori_loop(1, 10, cbody, (m0, lab0))
        cll[...] = lab
        pltpu.sync_copy(cll, cl_out)

    pltpu.sync_copy(ts_hbm, tsv)

    def g_to_br(g):
        b = g // _GB
        t = g - b * _GB
        r = jnp.where(t == _GB - 1, 5000 - _R, t * _R)
        return b, pl.multiple_of(r, 8)

    def start_in(g, bx_buf, sem):
        b, r = g_to_br(g)
        pltpu.async_copy(bx_hbm.at[pl.ds((b * 5000 + r) * 4, _R * 4)],
                         bx_buf, sem)

    def wait_in(bx_buf, sem):
        pltpu.make_async_copy(bx_hbm.at[pl.ds(0, _R * 4)], bx_buf,
                              sem).wait()

    def wait_out(ob_, osem):
        pltpu.make_async_copy(bx_out.at[pl.ds(0, _R * 4)], ob_,
                              osem).wait()

    start_in(wid, bxA, semA)
    start_in(wid + _NW, bxB, semB)

    def slot(i, sl, bx_buf, ob_, sem, osem):
        tt = 2 * i + sl
        g = wid + _NW * tt

        @pl.when(g < _G)
        def _():
            b, r = g_to_br(g)
            row0 = b * 5000 + r
            wait_in(bx_buf, sem)

            @pl.when(tt >= 2)
            def _():
                wait_out(ob_, osem)

            zero = jnp.zeros((16,), jnp.int32)
            shv = plsc.load_gather(tsv, [zero + 2 * b]).astype(jnp.float32)
            swv = plsc.load_gather(tsv, [zero + 2 * b + 1]).astype(
                jnp.float32)
            one = jnp.float32(1.0)
            zf = jnp.float32(0.0)
            for k in range(4):
                b4 = base4 + 64 * k
                cx = plsc.load_gather(bx_buf, [b4])
                cy = plsc.load_gather(bx_buf, [b4 + 1])
                w = plsc.load_gather(bx_buf, [b4 + 2])
                h = plsc.load_gather(bx_buf, [b4 + 3])
                x0 = jnp.clip(cx - 0.5 * w, zf, one) * swv
                y0 = jnp.clip(cy - 0.5 * h, zf, one) * shv
                x1 = jnp.clip(cx + 0.5 * w, zf, one) * swv
                y1 = jnp.clip(cy + 0.5 * h, zf, one) * shv
                plsc.store_scatter(ob_, [b4], x0)
                plsc.store_scatter(ob_, [b4 + 1], y0)
                plsc.store_scatter(ob_, [b4 + 2], x1)
                plsc.store_scatter(ob_, [b4 + 3], y1)

            pltpu.async_copy(ob_, bx_out.at[pl.ds(row0 * 4, _R * 4)], osem)

            g2 = g + 2 * _NW

            @pl.when(g2 < _G)
            def _():
                start_in(g2, bx_buf, sem)

    def lbody(i, _):
        slot(i, 0, bxA, obA, semA, osemA)
        slot(i, 1, bxB, obB, semB, osemB)
        return 0

    lax.fori_loop(0, _T // 2, lbody, 0)
    wait_out(obA, osemA)
    wait_out(obB, osemB)


def _tc_body(logits_ref, scores_ref, labels_ref):
    nq = logits_ref.shape[1]
    pad = _QPAD - nq
    xt = logits_ref[0].T
    c_iota = jax.lax.broadcasted_iota(jnp.int32, xt.shape, 0)
    m = jnp.max(xt, axis=0)
    labels = jnp.min(jnp.where(xt == m[None, :], c_iota, 91), axis=0)
    s = jnp.sum(jnp.exp(xt), axis=0)
    scores = jnp.exp(m) / s
    scores_ref[0] = jnp.concatenate(
        [scores, jnp.zeros((pad,), jnp.float32)]).reshape(1, _QPAD)
    labels_ref[0] = jnp.concatenate(
        [labels, jnp.zeros((pad,), jnp.int32)]).reshape(1, _QPAD)


def kernel(pred_logits, pred_boxes, cls_logits, target_sizes):
    nb, nq, nc = pred_logits.shape

    mesh = plsc.VectorSubcoreMesh(core_axis_name="c", subcore_axis_name="s")
    sc_fn = functools.partial(
        pl.kernel,
        mesh=mesh,
        compiler_params=pltpu.CompilerParams(needs_layout_passes=False),
        out_type=[
            jax.ShapeDtypeStruct((nb * nq * 4,), jnp.float32),
            jax.ShapeDtypeStruct((nb,), jnp.int32),
        ],
        scratch_types=[
            pltpu.VMEM((_R * 4,), jnp.float32),
            pltpu.VMEM((_R * 4,), jnp.float32),
            pltpu.VMEM((_R * 4,), jnp.float32),
            pltpu.VMEM((_R * 4,), jnp.float32),
            pltpu.VMEM((32,), jnp.int32),
            pltpu.VMEM((160,), jnp.float32),
            pltpu.VMEM((16,), jnp.int32),
            pltpu.SemaphoreType.DMA,
            pltpu.SemaphoreType.DMA,
            pltpu.SemaphoreType.DMA,
            pltpu.SemaphoreType.DMA,
        ],
    )(_sc_boxes)
    scores, labels = pl.pallas_call(
        _tc_body,
        grid=(nb,),
        in_specs=[pl.BlockSpec((1, nq, nc), lambda i: (i, 0, 0))],
        out_specs=[
            pl.BlockSpec((1, 1, _QPAD), lambda i: (i, 0, 0)),
            pl.BlockSpec((1, 1, _QPAD), lambda i: (i, 0, 0)),
        ],
        out_shape=[
            jax.ShapeDtypeStruct((nb, 1, _QPAD), jnp.float32),
            jax.ShapeDtypeStruct((nb, 1, _QPAD), jnp.int32),
        ],
    )(pred_logits)

    boxes, cls2 = sc_fn(pred_boxes.reshape(-1), cls_logits.reshape(-1),
                        target_sizes.reshape(-1))

    return (scores[:, 0, :nq], labels[:, 0, :nq],
            boxes.reshape(nb, nq, 4), cls2)
